# baseline (speedup 1.0000x reference)
import functools

import jax
import jax.numpy as jnp
from jax import lax
from jax.experimental import pallas as pl
from jax.experimental.pallas import tpu as pltpu
from jax.experimental.pallas import tpu_sc as plsc

N = 10000
H = 128
E = 320000
NC = 2
NS = 16
NW = NC * NS
EW = E // NW
K = 128
PH = 2
PHW = EW // PH
FULL = PHW // K
TAIL = PHW - FULL * K
KD = 128
DFULL = EW // KD
DTAIL = EW - DFULL * KD
NBUF = 2
DEG_R = N + 2000

_mesh = plsc.VectorSubcoreMesh(
    core_axis_name="c", subcore_axis_name="s", num_cores=NC, num_subcores=NS)


@functools.partial(
    pl.kernel,
    out_type=jax.ShapeDtypeStruct((NC * N,), jnp.float32),
    mesh=_mesh,
    scratch_types=[
        pltpu.VMEM((EW,), jnp.int32),
        pltpu.VMEM((KD,), jnp.float32),
        pltpu.VMEM((2000,), jnp.float32),
        pltpu.VMEM_SHARED((DEG_R,), jnp.float32),
    ],
)
def _sc_degree(ei_hbm, ones_hbm, zeros_hbm, out_hbm,
               idx_v, ones_v, stage_v, deg_sh):
    c = lax.axis_index("c")
    s = lax.axis_index("s")
    wid = c * NS + s

    @pl.when(s == 0)
    def _():
        pltpu.sync_copy(zeros_hbm, stage_v)
        for t in range(DEG_R // 2000):
            pltpu.sync_copy(stage_v, deg_sh.at[pl.ds(t * 2000, 2000)])

    pltpu.sync_copy(ei_hbm.at[pl.ds(E + wid * EW, EW)], idx_v)
    pltpu.sync_copy(ones_hbm, ones_v)
    plsc.subcore_barrier()

    def body(j, carry):
        pltpu.sync_copy(ones_v, deg_sh.at[idx_v.at[pl.ds(j * KD, KD)]],
                        add=True)
        return carry

    lax.fori_loop(0, DFULL, body, 0)
    pltpu.sync_copy(ones_v.at[pl.ds(0, DTAIL)],
                    deg_sh.at[idx_v.at[pl.ds(DFULL * KD, DTAIL)]], add=True)
    plsc.subcore_barrier()

    @pl.when(s == 0)
    def _():
        for t in range(N // 2000):
            pltpu.sync_copy(deg_sh.at[pl.ds(t * 2000, 2000)], stage_v)
            pltpu.sync_copy(stage_v, out_hbm.at[pl.ds(c * N + t * 2000, 2000)])


@functools.partial(
    pl.kernel,
    out_type=jax.ShapeDtypeStruct((NC * N, H), jnp.float32),
    mesh=_mesh,
    scratch_types=[
        pltpu.VMEM((PHW,), jnp.int32),
        pltpu.VMEM((PHW,), jnp.int32),
        pltpu.VMEM((NBUF, K, H), jnp.float32),
        pltpu.VMEM_SHARED((N, H), jnp.float32),
        [pltpu.SemaphoreType.DMA] * NBUF,
    ],
)
def _sc_scatter(y_hbm, ei_hbm, zrows_hbm, out_hbm,
                src_v, dst_v, rows_v, acc_sh, gsems):
    c = lax.axis_index("c")
    s = lax.axis_index("s")
    wid = c * NS + s

    @pl.when(s < NS - 1)
    def _():
        pltpu.sync_copy(zrows_hbm, acc_sh.at[pl.ds(s * 632, 632)])

    @pl.when(s == NS - 1)
    def _():
        pltpu.sync_copy(zrows_hbm.at[pl.ds(0, 520)],
                        acc_sh.at[pl.ds(15 * 632, 520)])

    plsc.subcore_barrier()

    def _gather(m, slot):
        pltpu.async_copy(y_hbm.at[src_v.at[pl.ds(m * K, K)]],
                         rows_v.at[slot], gsems[slot])

    def _gwait(m, slot):
        pltpu.make_async_copy(y_hbm.at[src_v.at[pl.ds(m * K, K)]],
                              rows_v.at[slot], gsems[slot]).wait()

    def _step(j, i, issue_next):
        _gwait(j, i)
        pltpu.sync_copy(rows_v.at[i],
                        acc_sh.at[dst_v.at[pl.ds(j * K, K)]], add=True)
        if issue_next:
            _gather(j + NBUF, i)

    for p in range(PH):
        base = (wid * PH + p) * PHW
        pltpu.sync_copy(ei_hbm.at[pl.ds(base, PHW)], src_v)
        pltpu.sync_copy(ei_hbm.at[pl.ds(E + base, PHW)], dst_v)
        for i in range(NBUF):
            _gather(i, i)

        def body(g, carry):
            for i in range(NBUF):
                _step(g * NBUF + i, i, True)
            return carry

        n_main = (FULL - 3) // NBUF
        lax.fori_loop(0, n_main, body, 0)
        for j in range(n_main * NBUF, FULL):
            _step(j, j % NBUF, j + NBUF < FULL)
        pltpu.async_copy(y_hbm.at[src_v.at[pl.ds(FULL * K, TAIL)]],
                         rows_v.at[0, pl.ds(0, TAIL)], gsems[0]).wait()
        pltpu.sync_copy(rows_v.at[0, pl.ds(0, TAIL)],
                        acc_sh.at[dst_v.at[pl.ds(FULL * K, TAIL)]], add=True)
    plsc.subcore_barrier()

    @pl.when(s < NS - 1)
    def _():
        pltpu.sync_copy(acc_sh.at[pl.ds(s * 632, 632)],
                        out_hbm.at[pl.ds(c * N + s * 632, 632)])

    @pl.when(s == NS - 1)
    def _():
        pltpu.sync_copy(acc_sh.at[pl.ds(15 * 632, 520)],
                        out_hbm.at[pl.ds(c * N + 15 * 632, 520)])


BR = 2000


def _tc_prescale_body(x_ref, w_ref, b_ref, d0_ref, d1_ref,
                      y_ref, r_ref, dis_ref):
    xw = jnp.dot(x_ref[...], w_ref[...], preferred_element_type=jnp.float32)
    degt = d0_ref[...] + d1_ref[...] + 1.0
    dis = lax.rsqrt(degt)
    y_ref[...] = xw * dis
    r_ref[...] = xw / degt + x_ref[...] + b_ref[...]
    dis_ref[...] = dis


def _tc_prescale(x, W, b2, deg2):
    grid = (N // BR,)
    return pl.pallas_call(
        _tc_prescale_body,
        grid=grid,
        in_specs=[
            pl.BlockSpec((BR, H), lambda i: (i, 0)),
            pl.BlockSpec((H, H), lambda i: (0, 0)),
            pl.BlockSpec((1, H), lambda i: (0, 0)),
            pl.BlockSpec((BR, 1), lambda i: (i, 0)),
            pl.BlockSpec((BR, 1), lambda i: (N // BR + i, 0)),
        ],
        out_specs=[
            pl.BlockSpec((BR, H), lambda i: (i, 0)),
            pl.BlockSpec((BR, H), lambda i: (i, 0)),
            pl.BlockSpec((BR, 1), lambda i: (i, 0)),
        ],
        out_shape=[
            jax.ShapeDtypeStruct((N, H), jnp.float32),
            jax.ShapeDtypeStruct((N, H), jnp.float32),
            jax.ShapeDtypeStruct((N, 1), jnp.float32),
        ],
    )(x, W, b2, deg2, deg2)


def _tc_finish_body(p0_ref, p1_ref, r_ref, dis_ref, o_ref):
    h = dis_ref[...] * (p0_ref[...] + p1_ref[...]) + r_ref[...]
    mean = jnp.mean(h, axis=1, keepdims=True)
    cent = h - mean
    var = jnp.mean(cent * cent, axis=1, keepdims=True)
    o_ref[...] = cent * lax.rsqrt(var + 1e-5)


def _tc_finish(acc, r, dis):
    grid = (N // BR,)
    return pl.pallas_call(
        _tc_finish_body,
        grid=grid,
        in_specs=[
            pl.BlockSpec((BR, H), lambda i: (i, 0)),
            pl.BlockSpec((BR, H), lambda i: (N // BR + i, 0)),
            pl.BlockSpec((BR, H), lambda i: (i, 0)),
            pl.BlockSpec((BR, 1), lambda i: (i, 0)),
        ],
        out_specs=pl.BlockSpec((BR, H), lambda i: (i, 0)),
        out_shape=jax.ShapeDtypeStruct((N, H), jnp.float32),
    )(acc, acc, r, dis)


def kernel(x, edge_index, batch, W, b):
    ei = edge_index.astype(jnp.int32).reshape(2 * E)

    ones_k = jnp.ones((KD,), jnp.float32)
    zeros_2k = jnp.zeros((2000,), jnp.float32)
    zrows = jnp.zeros((632, H), jnp.float32)

    deg = _sc_degree(ei, ones_k, zeros_2k)

    y, r, dis = _tc_prescale(x, W, b.reshape(1, H), deg.reshape(NC * N, 1))

    acc = _sc_scatter(y, ei, zrows)

    return _tc_finish(acc, r, dis)
